# initial kernel scaffold (unmeasured)
import jax
import jax.numpy as jnp
from jax import lax
from jax.experimental import pallas as pl
from jax.experimental.pallas import tpu as pltpu


def kernel(
    x,
):
    def body(*refs):
        pass

    out_shape = jax.ShapeDtypeStruct(..., jnp.float32)
    return pl.pallas_call(body, out_shape=out_shape)(...)


# baseline (device time: 13907 ns/iter reference)
import jax
import jax.numpy as jnp
from jax import lax
from jax.experimental import pallas as pl
from jax.experimental.pallas import tpu as pltpu

N_DEV = 8


def _local_cumprod(x):
    m, n = x.shape
    d = 1
    while d < m:
        shifted = jnp.concatenate(
            [jnp.ones((d, n), x.dtype), x[: m - d, :]], axis=0
        )
        x = x * shifted
        d *= 2
    return x


def kernel(x):
    m, n = x.shape

    def body(x_ref, out_ref, prefix_ref, total_ref, send_sem, recv_sem):
        my = lax.axis_index("i")

        loc = _local_cumprod(x_ref[...])
        out_ref[...] = loc
        total_ref[0, :] = loc[m - 1, :]

        @pl.when(my == 0)
        def _():
            prefix_ref[0, :] = jnp.ones((n,), jnp.float32)

        @pl.when(my > 0)
        def _():
            recv = pltpu.make_async_remote_copy(
                src_ref=total_ref,
                dst_ref=prefix_ref,
                send_sem=send_sem,
                recv_sem=recv_sem,
                device_id=(my - 1,),
                device_id_type=pl.DeviceIdType.MESH,
            )
            recv.wait_recv()

        @pl.when(my < N_DEV - 1)
        def _():
            total_ref[0, :] = total_ref[0, :] * prefix_ref[0, :]
            send = pltpu.make_async_remote_copy(
                src_ref=total_ref,
                dst_ref=prefix_ref,
                send_sem=send_sem,
                recv_sem=recv_sem,
                device_id=(my + 1,),
                device_id_type=pl.DeviceIdType.MESH,
            )
            send.start()
            send.wait_send()

        out_ref[...] = out_ref[...] * prefix_ref[0:1, :]

    return pl.pallas_call(
        body,
        out_shape=jax.ShapeDtypeStruct((m, n), jnp.float32),
        in_specs=[pl.BlockSpec(memory_space=pltpu.VMEM)],
        out_specs=pl.BlockSpec(memory_space=pltpu.VMEM),
        scratch_shapes=[
            pltpu.VMEM((1, n), jnp.float32),
            pltpu.VMEM((1, n), jnp.float32),
            pltpu.SemaphoreType.DMA,
            pltpu.SemaphoreType.DMA,
        ],
    )(x)


# device time: 12817 ns/iter; 1.0850x vs baseline; 1.0850x over previous
import jax
import jax.numpy as jnp
from jax import lax
from jax.experimental import pallas as pl
from jax.experimental.pallas import tpu as pltpu

N_DEV = 8
N_STEPS = 3


def _tree_prod(x):
    while x.shape[0] > 1:
        h = x.shape[0] // 2
        x = x[:h, :] * x[h:, :]
    return x


def _local_cumprod(x):
    m, n = x.shape
    d = 1
    while d < m:
        shifted = jnp.concatenate(
            [jnp.ones((d, n), x.dtype), x[: m - d, :]], axis=0
        )
        x = x * shifted
        d *= 2
    return x


def kernel(x):
    m, n = x.shape

    def body(x_ref, out_ref, incl_ref, excl_ref, recv_ref, send_sems, recv_sems):
        my = lax.axis_index("i")

        incl_ref[0:1, :] = _tree_prod(x_ref[...])
        excl_ref[0, :] = jnp.ones((n,), jnp.float32)

        for s in range(N_STEPS):
            d = 1 << s
            sends = my + d < N_DEV
            recvs = my - d >= 0
            copy = pltpu.make_async_remote_copy(
                src_ref=incl_ref,
                dst_ref=recv_ref.at[s],
                send_sem=send_sems.at[s],
                recv_sem=recv_sems.at[s],
                device_id=(jnp.minimum(my + d, N_DEV - 1),),
                device_id_type=pl.DeviceIdType.MESH,
            )

            @pl.when(sends)
            def _():
                copy.start()

            @pl.when(recvs)
            def _():
                copy.wait_recv()

            @pl.when(sends)
            def _():
                copy.wait_send()

            @pl.when(recvs)
            def _():
                r = recv_ref[s, 0, :]
                incl_ref[0, :] = incl_ref[0, :] * r
                excl_ref[0, :] = excl_ref[0, :] * r

        xv = x_ref[...]
        xv = jnp.concatenate([xv[0:1, :] * excl_ref[0:1, :], xv[1:, :]], axis=0)
        out_ref[...] = _local_cumprod(xv)

    return pl.pallas_call(
        body,
        out_shape=jax.ShapeDtypeStruct((m, n), jnp.float32),
        in_specs=[pl.BlockSpec(memory_space=pltpu.VMEM)],
        out_specs=pl.BlockSpec(memory_space=pltpu.VMEM),
        scratch_shapes=[
            pltpu.VMEM((1, n), jnp.float32),
            pltpu.VMEM((1, n), jnp.float32),
            pltpu.VMEM((N_STEPS, 1, n), jnp.float32),
            pltpu.SemaphoreType.DMA((N_STEPS,)),
            pltpu.SemaphoreType.DMA((N_STEPS,)),
        ],
    )(x)
